# baseline (speedup 1.0000x reference)
import functools

import jax
import jax.numpy as jnp
from jax import lax
from jax.experimental import pallas as pl
from jax.experimental.pallas import tpu as pltpu
from jax.experimental.pallas import tpu_sc as plsc

HIDDEN_SIZE = 512
NUM_SCENARIOS = 8
BATCH = 4096
NUM_CORES = 2
NUM_SUBCORES = 16
NUM_WORKERS = NUM_CORES * NUM_SUBCORES
B_PER_W = BATCH // NUM_WORKERS
LANES = 16
VPR = HIDDEN_SIZE // LANES

_mesh = plsc.VectorSubcoreMesh(core_axis_name="c", subcore_axis_name="s")


@functools.partial(
    pl.kernel,
    mesh=_mesh,
    out_type=jax.ShapeDtypeStruct((BATCH, HIDDEN_SIZE), jnp.float32),
    scratch_types=[
        pltpu.VMEM((B_PER_W,), jnp.int32),
        pltpu.VMEM((NUM_SCENARIOS, HIDDEN_SIZE), jnp.float32),
        pltpu.VMEM((B_PER_W, HIDDEN_SIZE), jnp.float32),
    ],
)
def _gather_kernel(idx_hbm, table_hbm, out_hbm, idx_v, tbl_v, rows_v):
    wid = lax.axis_index("s") * NUM_CORES + lax.axis_index("c")
    base = wid * B_PER_W
    pltpu.sync_copy(idx_hbm.at[pl.ds(base, B_PER_W)], idx_v)
    pltpu.sync_copy(table_hbm, tbl_v)

    def body(g, _):
        j0 = g * LANES
        idx_vec = idx_v[pl.ds(j0, LANES)]
        for l in range(LANES):
            r = idx_vec[l]
            for c in range(VPR):
                rows_v[j0 + l, pl.ds(c * LANES, LANES)] = tbl_v[r, pl.ds(c * LANES, LANES)]
        return 0

    lax.fori_loop(0, B_PER_W // LANES, body, 0)
    pltpu.sync_copy(rows_v, out_hbm.at[pl.ds(base, B_PER_W)])


def kernel(scenarios, table):
    return _gather_kernel(scenarios.astype(jnp.int32), table)


# speedup vs baseline: 1.2848x; 1.2848x over previous
import functools

import jax
import jax.numpy as jnp
from jax import lax
from jax.experimental import pallas as pl
from jax.experimental.pallas import tpu as pltpu
from jax.experimental.pallas import tpu_sc as plsc

HIDDEN_SIZE = 512
NUM_SCENARIOS = 8
BATCH = 4096
NUM_CORES = 2
NUM_SUBCORES = 16
NUM_WORKERS = NUM_CORES * NUM_SUBCORES
B_PER_W = BATCH // NUM_WORKERS
LANES = 16
NGROUP = B_PER_W // LANES

_mesh = plsc.VectorSubcoreMesh(core_axis_name="c", subcore_axis_name="s")


@functools.partial(
    pl.kernel,
    mesh=_mesh,
    out_type=jax.ShapeDtypeStruct((BATCH, HIDDEN_SIZE), jnp.float32),
    scratch_types=[
        pltpu.VMEM((B_PER_W,), jnp.int32),
        pltpu.VMEM((NUM_SCENARIOS, HIDDEN_SIZE), jnp.float32),
        pltpu.SemaphoreType.DMA,
    ],
)
def _gather_kernel(idx_hbm, table_hbm, out_hbm, idx_v, tbl_v, sem):
    wid = lax.axis_index("s") * NUM_CORES + lax.axis_index("c")
    base = wid * B_PER_W
    pltpu.sync_copy(idx_hbm.at[pl.ds(base, B_PER_W)], idx_v)
    pltpu.sync_copy(table_hbm, tbl_v)

    copies = []
    for g in range(NGROUP):
        idx_vec = idx_v[pl.ds(g * LANES, LANES)]
        for l in range(LANES):
            r = idx_vec[l]
            j = g * LANES + l
            copies.append(
                pltpu.async_copy(tbl_v.at[r], out_hbm.at[base + j], sem)
            )
    for cp in copies:
        cp.wait()


def kernel(scenarios, table):
    return _gather_kernel(scenarios.astype(jnp.int32), table)
